# Initial kernel scaffold; baseline (speedup 1.0000x reference)
import jax
import jax.numpy as jnp
from jax.experimental import pallas as pl


def kernel(x, edge_index, W_proj, b_proj, basis1, coeff1, bias1, basis2, coeff2, bias2, prompt_w):
    raise NotImplementedError("write your pallas kernel here")


# speedup vs baseline: 6.4481x; 6.4481x over previous
import functools

import jax
import jax.numpy as jnp
from jax import lax
from jax.experimental import pallas as pl
from jax.experimental.pallas import tpu as pltpu
from jax.experimental.pallas import tpu_sc as plsc

N = 10000
D = 128
E = 320000
R = 4
NB = 2
NC = 2
NS = 16
NPAD = 10240
TSPAN = NPAD // NS
EPT = E // R // NS
IDXW = 128
ROWS = 40
EPTP = ROWS * IDXW
PADE = EPTP - EPT
ZROWS = 64
BLK = 1024
GRID = NPAD // BLK

_sc_mesh = plsc.VectorSubcoreMesh(
    core_axis_name="c", subcore_axis_name="s", num_cores=NC, num_subcores=NS)


def _fill_rows(ref, rows, value):
  vec = jnp.full((16,), value, jnp.float32)

  def body(i, carry):
    for j in range(D // 16):
      ref[i, pl.ds(j * 16, 16)] = vec
    return carry

  lax.fori_loop(0, rows, body, 0)


@functools.partial(
    pl.kernel,
    out_type=jax.ShapeDtypeStruct((R * NPAD, D), jnp.float32),
    mesh=_sc_mesh,
    scratch_types=[
        pltpu.VMEM_SHARED((NPAD, D), jnp.float32),
        pltpu.VMEM((ROWS, IDXW), jnp.int32),
        pltpu.VMEM((ROWS, IDXW), jnp.int32),
        pltpu.VMEM((IDXW, D), jnp.float32),
        pltpu.VMEM((ZROWS, D), jnp.float32),
        pltpu.SemaphoreType.DMA,
    ],
)
def _sc_scatter(xw, srcg, dstg, out, acc, src_v, dst_v, rows_v, zbuf, gsem):
  c = lax.axis_index("c")
  s = lax.axis_index("s")
  _fill_rows(zbuf, ZROWS, 0.0)
  base = s * TSPAN
  for ri in range(R // NC):
    r = c * (R // NC) + ri
    for k in range(TSPAN // ZROWS):
      pltpu.sync_copy(zbuf, acc.at[pl.ds(base + k * ZROWS, ZROWS)])
    plsc.subcore_barrier()
    pltpu.sync_copy(srcg.at[r * NS + s], src_v)
    pltpu.sync_copy(dstg.at[r * NS + s], dst_v)

    def chunk(j, carry):
      pltpu.async_copy(xw.at[src_v.at[j]], rows_v, gsem).wait()
      pltpu.sync_copy(rows_v, acc.at[dst_v.at[j]], add=True)
      return carry

    lax.fori_loop(0, ROWS, chunk, 0)
    plsc.subcore_barrier()
    pltpu.sync_copy(acc.at[pl.ds(base, TSPAN)],
                    out.at[pl.ds(r * NPAD + base, TSPAN)])
    plsc.subcore_barrier()


@functools.partial(
    pl.kernel,
    out_type=jax.ShapeDtypeStruct((NC * NPAD, D), jnp.float32),
    mesh=_sc_mesh,
    scratch_types=[
        pltpu.VMEM_SHARED((NPAD, D), jnp.float32),
        pltpu.VMEM((ROWS, IDXW), jnp.int32),
        pltpu.VMEM((ROWS, IDXW), jnp.int32),
        pltpu.VMEM((IDXW, D), jnp.float32),
        pltpu.VMEM((ZROWS, D), jnp.float32),
        pltpu.SemaphoreType.DMA,
    ],
)
def _sc_final(hp, srcp, dstp, out, acc, src_v, dst_v, rows_v, zbuf, gsem):
  c = lax.axis_index("c")
  s = lax.axis_index("s")
  _fill_rows(zbuf, ZROWS, 0.0)
  base = s * TSPAN
  for k in range(TSPAN // ZROWS):
    pltpu.sync_copy(zbuf, acc.at[pl.ds(base + k * ZROWS, ZROWS)])
  plsc.subcore_barrier()
  for ri in range(R // NC):
    r = c * (R // NC) + ri
    pltpu.sync_copy(srcp.at[r * NS + s], src_v)
    pltpu.sync_copy(dstp.at[r * NS + s], dst_v)

    def chunk(j, carry):
      pltpu.async_copy(hp.at[src_v.at[j]], rows_v, gsem).wait()
      pltpu.sync_copy(rows_v, acc.at[dst_v.at[j]], add=True)
      return carry

    lax.fori_loop(0, ROWS, chunk, 0)
  plsc.subcore_barrier()
  pltpu.sync_copy(acc.at[pl.ds(base, TSPAN)],
                  out.at[pl.ds(c * NPAD + base, TSPAN)])


@functools.partial(
    pl.kernel,
    out_type=jax.ShapeDtypeStruct((R * NPAD,), jnp.float32),
    mesh=_sc_mesh,
    scratch_types=[
        pltpu.VMEM_SHARED(((R // NC) * NPAD,), jnp.float32),
        pltpu.VMEM((ROWS, IDXW), jnp.int32),
        pltpu.VMEM((ROWS, IDXW), jnp.float32),
        pltpu.VMEM(((R // NC) * NPAD // NS,), jnp.float32),
    ],
)
def _sc_deg(dstdeg, out, accd, idx_v, ones_v, zb):
  c = lax.axis_index("c")
  s = lax.axis_index("s")
  span = (R // NC) * NPAD // NS
  vec1 = jnp.full((16,), 1.0, jnp.float32)
  vec0 = jnp.zeros((16,), jnp.float32)

  def fill(i, carry):
    for j in range(IDXW // 16):
      ones_v[i, pl.ds(j * 16, 16)] = vec1
    return carry

  lax.fori_loop(0, ROWS, fill, 0)

  def fillz(i, carry):
    zb[pl.ds(i * 16, 16)] = vec0
    return carry

  lax.fori_loop(0, span // 16, fillz, 0)
  pltpu.sync_copy(zb, accd.at[pl.ds(s * span, span)])
  plsc.subcore_barrier()
  for ri in range(R // NC):
    r = c * (R // NC) + ri
    pltpu.sync_copy(dstdeg.at[r * NS + s], idx_v)

    def chunk(j, carry):
      pltpu.sync_copy(ones_v.at[j], accd.at[idx_v.at[j]], add=True)
      return carry

    lax.fori_loop(0, ROWS, chunk, 0)
  plsc.subcore_barrier()
  pltpu.sync_copy(accd.at[pl.ds(s * span, span)],
                  out.at[pl.ds(c * (R // NC) * NPAD + s * span, span)])


def _tc_proj_body(x_ref, wp_ref, bp_ref, basis_ref, coeff_ref, out_ref):
  h0 = jnp.dot(x_ref[...], wp_ref[...],
               preferred_element_type=jnp.float32) + bp_ref[...]
  for r in range(R):
    w = basis_ref[0] * coeff_ref[r, 0] + basis_ref[1] * coeff_ref[r, 1]
    out_ref[r] = jnp.dot(h0, w, preferred_element_type=jnp.float32)


_tc_proj = pl.pallas_call(
    _tc_proj_body,
    grid=(GRID,),
    in_specs=[
        pl.BlockSpec((BLK, D), lambda i: (i, 0)),
        pl.BlockSpec((D, D), lambda i: (0, 0)),
        pl.BlockSpec((1, D), lambda i: (0, 0)),
        pl.BlockSpec((NB, D, D), lambda i: (0, 0, 0)),
        pl.BlockSpec((R, NB), lambda i: (0, 0)),
    ],
    out_specs=pl.BlockSpec((R, BLK, D), lambda i: (0, i, 0)),
    out_shape=jax.ShapeDtypeStruct((R, NPAD, D), jnp.float32),
)


def _norm_relu(s_ref, deg_ref, bias_ref):
  invd = 1.0 / jnp.maximum(deg_ref[...], 1.0)
  h = s_ref[0] * invd[0][:, None]
  for r in range(1, R):
    h = h + s_ref[r] * invd[r][:, None]
  return jnp.maximum(h + bias_ref[...], 0.0)


def _tc_conv_body(s_ref, deg_ref, bias_ref, basis_ref, coeff_ref, out_ref):
  h = _norm_relu(s_ref, deg_ref, bias_ref)
  for r in range(R):
    w = basis_ref[0] * coeff_ref[r, 0] + basis_ref[1] * coeff_ref[r, 1]
    out_ref[r] = jnp.dot(h, w, preferred_element_type=jnp.float32)


_tc_conv = pl.pallas_call(
    _tc_conv_body,
    grid=(GRID,),
    in_specs=[
        pl.BlockSpec((R, BLK, D), lambda i: (0, i, 0)),
        pl.BlockSpec((R, BLK), lambda i: (0, i)),
        pl.BlockSpec((1, D), lambda i: (0, 0)),
        pl.BlockSpec((NB, D, D), lambda i: (0, 0, 0)),
        pl.BlockSpec((R, NB), lambda i: (0, 0)),
    ],
    out_specs=pl.BlockSpec((R, BLK, D), lambda i: (0, i, 0)),
    out_shape=jax.ShapeDtypeStruct((R, NPAD, D), jnp.float32),
)


def _tc_prompt_body(s_ref, deg_ref, bias_ref, pw_ref, out_ref):
  h = _norm_relu(s_ref, deg_ref, bias_ref)
  z = h * pw_ref[...]
  out_ref[...] = jnp.where(z > 0, z, jnp.exp(jnp.minimum(z, 0.0)) - 1.0)


_tc_prompt = pl.pallas_call(
    _tc_prompt_body,
    grid=(GRID,),
    in_specs=[
        pl.BlockSpec((R, BLK, D), lambda i: (0, i, 0)),
        pl.BlockSpec((R, BLK), lambda i: (0, i)),
        pl.BlockSpec((1, D), lambda i: (0, 0)),
        pl.BlockSpec((1, D), lambda i: (0, 0)),
    ],
    out_specs=pl.BlockSpec((BLK, D), lambda i: (i, 0)),
    out_shape=jax.ShapeDtypeStruct((NPAD, D), jnp.float32),
)


def _tc_add_body(p_ref, out_ref):
  out_ref[...] = p_ref[0] + p_ref[1]


_tc_add = pl.pallas_call(
    _tc_add_body,
    grid=(GRID,),
    in_specs=[pl.BlockSpec((NC, BLK, D), lambda i: (0, i, 0))],
    out_specs=pl.BlockSpec((BLK, D), lambda i: (i, 0)),
    out_shape=jax.ShapeDtypeStruct((N, D), jnp.float32),
)


def kernel(x, edge_index, W_proj, b_proj, basis1, coeff1, bias1,
           basis2, coeff2, bias2, prompt_w):
  f32 = jnp.float32
  x_pad = jnp.concatenate([x, jnp.zeros((NPAD - N, D), f32)], axis=0)

  src = edge_index[0].astype(jnp.int32).reshape(R, NS, EPT)
  dst = edge_index[1].astype(jnp.int32).reshape(R, NS, EPT)
  padv = (jnp.arange(PADE, dtype=jnp.int32) % 8)
  pad_src = jnp.broadcast_to(padv, (R, NS, PADE))
  pad_dst = pad_src + N
  src3 = jnp.concatenate([src, pad_src], axis=2)
  dst3 = jnp.concatenate([dst, pad_dst], axis=2)
  roff = (jnp.arange(R, dtype=jnp.int32) * NPAD)[:, None, None]
  doff = ((jnp.arange(R, dtype=jnp.int32) % NC) * NPAD)[:, None, None]
  src_p = src3.reshape(R * NS, ROWS, IDXW)
  srcg = (src3 + roff).reshape(R * NS, ROWS, IDXW)
  dst_p = dst3.reshape(R * NS, ROWS, IDXW)
  dstdeg = (dst3 + doff).reshape(R * NS, ROWS, IDXW)

  b_proj2 = b_proj.reshape(1, D)
  bias1_2 = bias1.reshape(1, D)
  bias2_2 = bias2.reshape(1, D)
  pw2 = prompt_w.reshape(1, D)

  deg = _sc_deg(dstdeg).reshape(R, NPAD)
  xw1 = _tc_proj(x_pad, W_proj, b_proj2, basis1, coeff1)
  S1 = _sc_scatter(xw1.reshape(R * NPAD, D), srcg, dst_p).reshape(R, NPAD, D)
  xw2 = _tc_conv(S1, deg, bias1_2, basis2, coeff2)
  S2 = _sc_scatter(xw2.reshape(R * NPAD, D), srcg, dst_p).reshape(R, NPAD, D)
  hp = _tc_prompt(S2, deg, bias2_2, pw2)
  Pf = _sc_final(hp, src_p, dst_p)
  return _tc_add(Pf.reshape(NC, NPAD, D))


# speedup vs baseline: 7.4752x; 1.1593x over previous
import functools

import jax
import jax.numpy as jnp
from jax import lax
from jax.experimental import pallas as pl
from jax.experimental.pallas import tpu as pltpu
from jax.experimental.pallas import tpu_sc as plsc

N = 10000
D = 128
E = 320000
R = 4
NB = 2
NC = 2
NS = 16
NPAD = 10240
TSPAN = NPAD // NS
EPT = E // R // NS
IDXW = 128
ROWS = 40
EPTP = ROWS * IDXW
PADE = EPTP - EPT
ZROWS = 64
BLK = 1024
GRID = NPAD // BLK

_sc_mesh = plsc.VectorSubcoreMesh(
    core_axis_name="c", subcore_axis_name="s", num_cores=NC, num_subcores=NS)


NBUF = 2


def _fill_rows(ref, rows, value):
  vec = jnp.full((16,), value, jnp.float32)

  def body(i, carry):
    for j in range(D // 16):
      ref[i, pl.ds(j * 16, 16)] = vec
    return carry

  lax.fori_loop(0, rows, body, 0)


def _edge_pass(xw, acc, src_v, dst_v, bufs, gsems, ssems):
  for b in range(NBUF):
    pltpu.async_copy(xw.at[src_v.at[b]], bufs[b], gsems[b])

  def body(i, carry):
    for b in range(NBUF):
      pltpu.make_async_copy(xw.at[pl.ds(0, IDXW)], bufs[b], gsems[b]).wait()
      pltpu.async_copy(bufs[b], acc.at[dst_v.at[NBUF * i + b]], ssems[b],
                       add=True)
    for b in range(NBUF):
      pltpu.make_async_copy(bufs[b], acc.at[pl.ds(0, IDXW)], ssems[b]).wait()

      @pl.when(i < ROWS // NBUF - 1)
      def _():
        pltpu.async_copy(xw.at[src_v.at[NBUF * (i + 1) + b]], bufs[b],
                         gsems[b])

    return carry

  lax.fori_loop(0, ROWS // NBUF, body, 0)


@functools.partial(
    pl.kernel,
    out_type=jax.ShapeDtypeStruct((R * NPAD, D), jnp.float32),
    mesh=_sc_mesh,
    scratch_types=[
        pltpu.VMEM_SHARED((NPAD, D), jnp.float32),
        pltpu.VMEM((ROWS, IDXW), jnp.int32),
        pltpu.VMEM((ROWS, IDXW), jnp.int32),
        [pltpu.VMEM((IDXW, D), jnp.float32)] * NBUF,
        [pltpu.SemaphoreType.DMA] * NBUF,
        [pltpu.SemaphoreType.DMA] * NBUF,
    ],
)
def _sc_scatter(xw, srcg, dstg, out, acc, src_v, dst_v, bufs, gsems,
                ssems):
  c = lax.axis_index("c")
  s = lax.axis_index("s")
  base = s * TSPAN
  for ri in range(R // NC):
    r = c * (R // NC) + ri
    _fill_rows(bufs[0], IDXW, 0.0)
    for k in range(TSPAN // IDXW):
      pltpu.sync_copy(bufs[0], acc.at[pl.ds(base + k * IDXW, IDXW)])
    plsc.subcore_barrier()
    pltpu.sync_copy(srcg.at[r * NS + s], src_v)
    pltpu.sync_copy(dstg.at[r * NS + s], dst_v)
    _edge_pass(xw, acc, src_v, dst_v, bufs, gsems, ssems)
    plsc.subcore_barrier()
    pltpu.sync_copy(acc.at[pl.ds(base, TSPAN)],
                    out.at[pl.ds(r * NPAD + base, TSPAN)])
    plsc.subcore_barrier()


@functools.partial(
    pl.kernel,
    out_type=jax.ShapeDtypeStruct((NC * NPAD, D), jnp.float32),
    mesh=_sc_mesh,
    scratch_types=[
        pltpu.VMEM_SHARED((NPAD, D), jnp.float32),
        pltpu.VMEM((ROWS, IDXW), jnp.int32),
        pltpu.VMEM((ROWS, IDXW), jnp.int32),
        [pltpu.VMEM((IDXW, D), jnp.float32)] * NBUF,
        [pltpu.SemaphoreType.DMA] * NBUF,
        [pltpu.SemaphoreType.DMA] * NBUF,
    ],
)
def _sc_final(hp, srcp, dstp, out, acc, src_v, dst_v, bufs, gsems,
              ssems):
  c = lax.axis_index("c")
  s = lax.axis_index("s")
  _fill_rows(bufs[0], IDXW, 0.0)
  base = s * TSPAN
  for k in range(TSPAN // IDXW):
    pltpu.sync_copy(bufs[0], acc.at[pl.ds(base + k * IDXW, IDXW)])
  plsc.subcore_barrier()
  for ri in range(R // NC):
    r = c * (R // NC) + ri
    pltpu.sync_copy(srcp.at[r * NS + s], src_v)
    pltpu.sync_copy(dstp.at[r * NS + s], dst_v)
    _edge_pass(hp, acc, src_v, dst_v, bufs, gsems, ssems)
  plsc.subcore_barrier()
  pltpu.sync_copy(acc.at[pl.ds(base, TSPAN)],
                  out.at[pl.ds(c * NPAD + base, TSPAN)])


@functools.partial(
    pl.kernel,
    out_type=jax.ShapeDtypeStruct((R * NPAD,), jnp.float32),
    mesh=_sc_mesh,
    scratch_types=[
        pltpu.VMEM_SHARED(((R // NC) * NPAD,), jnp.float32),
        pltpu.VMEM((ROWS, IDXW), jnp.int32),
        pltpu.VMEM((ROWS, IDXW), jnp.float32),
        pltpu.VMEM(((R // NC) * NPAD // NS,), jnp.float32),
    ],
)
def _sc_deg(dstdeg, out, accd, idx_v, ones_v, zb):
  c = lax.axis_index("c")
  s = lax.axis_index("s")
  span = (R // NC) * NPAD // NS
  vec1 = jnp.full((16,), 1.0, jnp.float32)
  vec0 = jnp.zeros((16,), jnp.float32)

  def fill(i, carry):
    for j in range(IDXW // 16):
      ones_v[i, pl.ds(j * 16, 16)] = vec1
    return carry

  lax.fori_loop(0, ROWS, fill, 0)

  def fillz(i, carry):
    zb[pl.ds(i * 16, 16)] = vec0
    return carry

  lax.fori_loop(0, span // 16, fillz, 0)
  pltpu.sync_copy(zb, accd.at[pl.ds(s * span, span)])
  plsc.subcore_barrier()
  for ri in range(R // NC):
    r = c * (R // NC) + ri
    pltpu.sync_copy(dstdeg.at[r * NS + s], idx_v)

    def chunk(j, carry):
      pltpu.sync_copy(ones_v.at[j], accd.at[idx_v.at[j]], add=True)
      return carry

    lax.fori_loop(0, ROWS, chunk, 0)
  plsc.subcore_barrier()
  pltpu.sync_copy(accd.at[pl.ds(s * span, span)],
                  out.at[pl.ds(c * (R // NC) * NPAD + s * span, span)])


def _tc_proj_body(x_ref, wp_ref, bp_ref, basis_ref, coeff_ref, out_ref):
  h0 = jnp.dot(x_ref[...], wp_ref[...],
               preferred_element_type=jnp.float32) + bp_ref[...]
  for r in range(R):
    w = basis_ref[0] * coeff_ref[r, 0] + basis_ref[1] * coeff_ref[r, 1]
    out_ref[r] = jnp.dot(h0, w, preferred_element_type=jnp.float32)


_tc_proj = pl.pallas_call(
    _tc_proj_body,
    grid=(GRID,),
    in_specs=[
        pl.BlockSpec((BLK, D), lambda i: (i, 0)),
        pl.BlockSpec((D, D), lambda i: (0, 0)),
        pl.BlockSpec((1, D), lambda i: (0, 0)),
        pl.BlockSpec((NB, D, D), lambda i: (0, 0, 0)),
        pl.BlockSpec((R, NB), lambda i: (0, 0)),
    ],
    out_specs=pl.BlockSpec((R, BLK, D), lambda i: (0, i, 0)),
    out_shape=jax.ShapeDtypeStruct((R, NPAD, D), jnp.float32),
)


def _norm_relu(s_ref, deg_ref, bias_ref):
  invd = 1.0 / jnp.maximum(deg_ref[...], 1.0)
  h = s_ref[0] * invd[0][:, None]
  for r in range(1, R):
    h = h + s_ref[r] * invd[r][:, None]
  return jnp.maximum(h + bias_ref[...], 0.0)


def _tc_conv_body(s_ref, deg_ref, bias_ref, basis_ref, coeff_ref, out_ref):
  h = _norm_relu(s_ref, deg_ref, bias_ref)
  for r in range(R):
    w = basis_ref[0] * coeff_ref[r, 0] + basis_ref[1] * coeff_ref[r, 1]
    out_ref[r] = jnp.dot(h, w, preferred_element_type=jnp.float32)


_tc_conv = pl.pallas_call(
    _tc_conv_body,
    grid=(GRID,),
    in_specs=[
        pl.BlockSpec((R, BLK, D), lambda i: (0, i, 0)),
        pl.BlockSpec((R, BLK), lambda i: (0, i)),
        pl.BlockSpec((1, D), lambda i: (0, 0)),
        pl.BlockSpec((NB, D, D), lambda i: (0, 0, 0)),
        pl.BlockSpec((R, NB), lambda i: (0, 0)),
    ],
    out_specs=pl.BlockSpec((R, BLK, D), lambda i: (0, i, 0)),
    out_shape=jax.ShapeDtypeStruct((R, NPAD, D), jnp.float32),
)


def _tc_prompt_body(s_ref, deg_ref, bias_ref, pw_ref, out_ref):
  h = _norm_relu(s_ref, deg_ref, bias_ref)
  z = h * pw_ref[...]
  out_ref[...] = jnp.where(z > 0, z, jnp.exp(jnp.minimum(z, 0.0)) - 1.0)


_tc_prompt = pl.pallas_call(
    _tc_prompt_body,
    grid=(GRID,),
    in_specs=[
        pl.BlockSpec((R, BLK, D), lambda i: (0, i, 0)),
        pl.BlockSpec((R, BLK), lambda i: (0, i)),
        pl.BlockSpec((1, D), lambda i: (0, 0)),
        pl.BlockSpec((1, D), lambda i: (0, 0)),
    ],
    out_specs=pl.BlockSpec((BLK, D), lambda i: (i, 0)),
    out_shape=jax.ShapeDtypeStruct((NPAD, D), jnp.float32),
)


def _tc_add_body(p_ref, out_ref):
  out_ref[...] = p_ref[0] + p_ref[1]


_tc_add = pl.pallas_call(
    _tc_add_body,
    grid=(GRID,),
    in_specs=[pl.BlockSpec((NC, BLK, D), lambda i: (0, i, 0))],
    out_specs=pl.BlockSpec((BLK, D), lambda i: (i, 0)),
    out_shape=jax.ShapeDtypeStruct((N, D), jnp.float32),
)


def kernel(x, edge_index, W_proj, b_proj, basis1, coeff1, bias1,
           basis2, coeff2, bias2, prompt_w):
  f32 = jnp.float32
  x_pad = jnp.concatenate([x, jnp.zeros((NPAD - N, D), f32)], axis=0)

  src = edge_index[0].astype(jnp.int32).reshape(R, NS, EPT)
  dst = edge_index[1].astype(jnp.int32).reshape(R, NS, EPT)
  padv = (jnp.arange(PADE, dtype=jnp.int32) % 8)
  pad_src = jnp.broadcast_to(padv, (R, NS, PADE))
  pad_dst = pad_src + N
  src3 = jnp.concatenate([src, pad_src], axis=2)
  dst3 = jnp.concatenate([dst, pad_dst], axis=2)
  roff = (jnp.arange(R, dtype=jnp.int32) * NPAD)[:, None, None]
  doff = ((jnp.arange(R, dtype=jnp.int32) % NC) * NPAD)[:, None, None]
  src_p = src3.reshape(R * NS, ROWS, IDXW)
  srcg = (src3 + roff).reshape(R * NS, ROWS, IDXW)
  dst_p = dst3.reshape(R * NS, ROWS, IDXW)
  dstdeg = (dst3 + doff).reshape(R * NS, ROWS, IDXW)

  b_proj2 = b_proj.reshape(1, D)
  bias1_2 = bias1.reshape(1, D)
  bias2_2 = bias2.reshape(1, D)
  pw2 = prompt_w.reshape(1, D)

  deg = _sc_deg(dstdeg).reshape(R, NPAD)
  xw1 = _tc_proj(x_pad, W_proj, b_proj2, basis1, coeff1)
  S1 = _sc_scatter(xw1.reshape(R * NPAD, D), srcg, dst_p).reshape(R, NPAD, D)
  xw2 = _tc_conv(S1, deg, bias1_2, basis2, coeff2)
  S2 = _sc_scatter(xw2.reshape(R * NPAD, D), srcg, dst_p).reshape(R, NPAD, D)
  hp = _tc_prompt(S2, deg, bias2_2, pw2)
  Pf = _sc_final(hp, src_p, dst_p)
  return _tc_add(Pf.reshape(NC, NPAD, D))


# speedup vs baseline: 7.5831x; 1.0144x over previous
import functools

import jax
import jax.numpy as jnp
from jax import lax
from jax.experimental import pallas as pl
from jax.experimental.pallas import tpu as pltpu
from jax.experimental.pallas import tpu_sc as plsc

N = 10000
D = 128
E = 320000
R = 4
NB = 2
NC = 2
NS = 16
NPAD = 10240
TSPAN = NPAD // NS
EPT = E // R // NS
IDXW = 64
ROWS = 81
EPTP = ROWS * IDXW
PADE = EPTP - EPT
ZROWS = 64
BLK = 1024
GRID = NPAD // BLK

_sc_mesh = plsc.VectorSubcoreMesh(
    core_axis_name="c", subcore_axis_name="s", num_cores=NC, num_subcores=NS)


NBUF = 3


def _fill_rows(ref, rows, value):
  vec = jnp.full((16,), value, jnp.float32)

  def body(i, carry):
    for j in range(D // 16):
      ref[i, pl.ds(j * 16, 16)] = vec
    return carry

  lax.fori_loop(0, rows, body, 0)


def _edge_pass(xw, acc, src_v, dst_v, bufs, gsems, ssems):
  for b in range(NBUF):
    pltpu.async_copy(xw.at[src_v.at[b]], bufs[b], gsems[b])

  def body(i, carry):
    for b in range(NBUF):
      pltpu.make_async_copy(xw.at[pl.ds(0, IDXW)], bufs[b], gsems[b]).wait()
      pltpu.async_copy(bufs[b], acc.at[dst_v.at[NBUF * i + b]], ssems[b],
                       add=True)
    for b in range(NBUF):
      pltpu.make_async_copy(bufs[b], acc.at[pl.ds(0, IDXW)], ssems[b]).wait()

      @pl.when(i < ROWS // NBUF - 1)
      def _():
        pltpu.async_copy(xw.at[src_v.at[NBUF * (i + 1) + b]], bufs[b],
                         gsems[b])

    return carry

  lax.fori_loop(0, ROWS // NBUF, body, 0)


@functools.partial(
    pl.kernel,
    out_type=jax.ShapeDtypeStruct((R * NPAD, D), jnp.float32),
    mesh=_sc_mesh,
    scratch_types=[
        pltpu.VMEM_SHARED((NPAD, D), jnp.float32),
        pltpu.VMEM((ROWS, IDXW), jnp.int32),
        pltpu.VMEM((ROWS, IDXW), jnp.int32),
        [pltpu.VMEM((IDXW, D), jnp.float32)] * NBUF,
        [pltpu.SemaphoreType.DMA] * NBUF,
        [pltpu.SemaphoreType.DMA] * NBUF,
    ],
)
def _sc_scatter(xw, srcg, dstg, out, acc, src_v, dst_v, bufs, gsems,
                ssems):
  c = lax.axis_index("c")
  s = lax.axis_index("s")
  base = s * TSPAN
  for ri in range(R // NC):
    r = c * (R // NC) + ri
    _fill_rows(bufs[0], IDXW, 0.0)
    for k in range(TSPAN // IDXW):
      pltpu.sync_copy(bufs[0], acc.at[pl.ds(base + k * IDXW, IDXW)])
    plsc.subcore_barrier()
    pltpu.sync_copy(srcg.at[r * NS + s], src_v)
    pltpu.sync_copy(dstg.at[r * NS + s], dst_v)
    _edge_pass(xw, acc, src_v, dst_v, bufs, gsems, ssems)
    plsc.subcore_barrier()
    pltpu.sync_copy(acc.at[pl.ds(base, TSPAN)],
                    out.at[pl.ds(r * NPAD + base, TSPAN)])
    plsc.subcore_barrier()


@functools.partial(
    pl.kernel,
    out_type=jax.ShapeDtypeStruct((NC * NPAD, D), jnp.float32),
    mesh=_sc_mesh,
    scratch_types=[
        pltpu.VMEM_SHARED((NPAD, D), jnp.float32),
        pltpu.VMEM((ROWS, IDXW), jnp.int32),
        pltpu.VMEM((ROWS, IDXW), jnp.int32),
        [pltpu.VMEM((IDXW, D), jnp.float32)] * NBUF,
        [pltpu.SemaphoreType.DMA] * NBUF,
        [pltpu.SemaphoreType.DMA] * NBUF,
    ],
)
def _sc_final(hp, srcp, dstp, out, acc, src_v, dst_v, bufs, gsems,
              ssems):
  c = lax.axis_index("c")
  s = lax.axis_index("s")
  _fill_rows(bufs[0], IDXW, 0.0)
  base = s * TSPAN
  for k in range(TSPAN // IDXW):
    pltpu.sync_copy(bufs[0], acc.at[pl.ds(base + k * IDXW, IDXW)])
  plsc.subcore_barrier()
  for ri in range(R // NC):
    r = c * (R // NC) + ri
    pltpu.sync_copy(srcp.at[r * NS + s], src_v)
    pltpu.sync_copy(dstp.at[r * NS + s], dst_v)
    _edge_pass(hp, acc, src_v, dst_v, bufs, gsems, ssems)
  plsc.subcore_barrier()
  pltpu.sync_copy(acc.at[pl.ds(base, TSPAN)],
                  out.at[pl.ds(c * NPAD + base, TSPAN)])


@functools.partial(
    pl.kernel,
    out_type=jax.ShapeDtypeStruct((R * NPAD,), jnp.float32),
    mesh=_sc_mesh,
    scratch_types=[
        pltpu.VMEM_SHARED(((R // NC) * NPAD,), jnp.float32),
        pltpu.VMEM((ROWS, IDXW), jnp.int32),
        pltpu.VMEM((ROWS, IDXW), jnp.float32),
        pltpu.VMEM(((R // NC) * NPAD // NS,), jnp.float32),
    ],
)
def _sc_deg(dstdeg, out, accd, idx_v, ones_v, zb):
  c = lax.axis_index("c")
  s = lax.axis_index("s")
  span = (R // NC) * NPAD // NS
  vec1 = jnp.full((16,), 1.0, jnp.float32)
  vec0 = jnp.zeros((16,), jnp.float32)

  def fill(i, carry):
    for j in range(IDXW // 16):
      ones_v[i, pl.ds(j * 16, 16)] = vec1
    return carry

  lax.fori_loop(0, ROWS, fill, 0)

  def fillz(i, carry):
    zb[pl.ds(i * 16, 16)] = vec0
    return carry

  lax.fori_loop(0, span // 16, fillz, 0)
  pltpu.sync_copy(zb, accd.at[pl.ds(s * span, span)])
  plsc.subcore_barrier()
  for ri in range(R // NC):
    r = c * (R // NC) + ri
    pltpu.sync_copy(dstdeg.at[r * NS + s], idx_v)

    def chunk(j, carry):
      pltpu.sync_copy(ones_v.at[j], accd.at[idx_v.at[j]], add=True)
      return carry

    lax.fori_loop(0, ROWS, chunk, 0)
  plsc.subcore_barrier()
  pltpu.sync_copy(accd.at[pl.ds(s * span, span)],
                  out.at[pl.ds(c * (R // NC) * NPAD + s * span, span)])


def _tc_proj_body(x_ref, wp_ref, bp_ref, basis_ref, coeff_ref, out_ref):
  h0 = jnp.dot(x_ref[...], wp_ref[...],
               preferred_element_type=jnp.float32) + bp_ref[...]
  for r in range(R):
    w = basis_ref[0] * coeff_ref[r, 0] + basis_ref[1] * coeff_ref[r, 1]
    out_ref[r] = jnp.dot(h0, w, preferred_element_type=jnp.float32)


_tc_proj = pl.pallas_call(
    _tc_proj_body,
    grid=(GRID,),
    in_specs=[
        pl.BlockSpec((BLK, D), lambda i: (i, 0)),
        pl.BlockSpec((D, D), lambda i: (0, 0)),
        pl.BlockSpec((1, D), lambda i: (0, 0)),
        pl.BlockSpec((NB, D, D), lambda i: (0, 0, 0)),
        pl.BlockSpec((R, NB), lambda i: (0, 0)),
    ],
    out_specs=pl.BlockSpec((R, BLK, D), lambda i: (0, i, 0)),
    out_shape=jax.ShapeDtypeStruct((R, NPAD, D), jnp.float32),
)


def _norm_relu(s_ref, deg_ref, bias_ref):
  invd = 1.0 / jnp.maximum(deg_ref[...], 1.0)
  h = s_ref[0] * invd[0][:, None]
  for r in range(1, R):
    h = h + s_ref[r] * invd[r][:, None]
  return jnp.maximum(h + bias_ref[...], 0.0)


def _tc_conv_body(s_ref, deg_ref, bias_ref, basis_ref, coeff_ref, out_ref):
  h = _norm_relu(s_ref, deg_ref, bias_ref)
  for r in range(R):
    w = basis_ref[0] * coeff_ref[r, 0] + basis_ref[1] * coeff_ref[r, 1]
    out_ref[r] = jnp.dot(h, w, preferred_element_type=jnp.float32)


_tc_conv = pl.pallas_call(
    _tc_conv_body,
    grid=(GRID,),
    in_specs=[
        pl.BlockSpec((R, BLK, D), lambda i: (0, i, 0)),
        pl.BlockSpec((R, BLK), lambda i: (0, i)),
        pl.BlockSpec((1, D), lambda i: (0, 0)),
        pl.BlockSpec((NB, D, D), lambda i: (0, 0, 0)),
        pl.BlockSpec((R, NB), lambda i: (0, 0)),
    ],
    out_specs=pl.BlockSpec((R, BLK, D), lambda i: (0, i, 0)),
    out_shape=jax.ShapeDtypeStruct((R, NPAD, D), jnp.float32),
)


def _tc_prompt_body(s_ref, deg_ref, bias_ref, pw_ref, out_ref):
  h = _norm_relu(s_ref, deg_ref, bias_ref)
  z = h * pw_ref[...]
  out_ref[...] = jnp.where(z > 0, z, jnp.exp(jnp.minimum(z, 0.0)) - 1.0)


_tc_prompt = pl.pallas_call(
    _tc_prompt_body,
    grid=(GRID,),
    in_specs=[
        pl.BlockSpec((R, BLK, D), lambda i: (0, i, 0)),
        pl.BlockSpec((R, BLK), lambda i: (0, i)),
        pl.BlockSpec((1, D), lambda i: (0, 0)),
        pl.BlockSpec((1, D), lambda i: (0, 0)),
    ],
    out_specs=pl.BlockSpec((BLK, D), lambda i: (i, 0)),
    out_shape=jax.ShapeDtypeStruct((NPAD, D), jnp.float32),
)


def _tc_add_body(p_ref, out_ref):
  out_ref[...] = p_ref[0] + p_ref[1]


_tc_add = pl.pallas_call(
    _tc_add_body,
    grid=(GRID,),
    in_specs=[pl.BlockSpec((NC, BLK, D), lambda i: (0, i, 0))],
    out_specs=pl.BlockSpec((BLK, D), lambda i: (i, 0)),
    out_shape=jax.ShapeDtypeStruct((N, D), jnp.float32),
)


def kernel(x, edge_index, W_proj, b_proj, basis1, coeff1, bias1,
           basis2, coeff2, bias2, prompt_w):
  f32 = jnp.float32
  x_pad = jnp.concatenate([x, jnp.zeros((NPAD - N, D), f32)], axis=0)

  src = edge_index[0].astype(jnp.int32).reshape(R, NS, EPT)
  dst = edge_index[1].astype(jnp.int32).reshape(R, NS, EPT)
  padv = (jnp.arange(PADE, dtype=jnp.int32) % 8)
  pad_src = jnp.broadcast_to(padv, (R, NS, PADE))
  pad_dst = pad_src + N
  src3 = jnp.concatenate([src, pad_src], axis=2)
  dst3 = jnp.concatenate([dst, pad_dst], axis=2)
  roff = (jnp.arange(R, dtype=jnp.int32) * NPAD)[:, None, None]
  doff = ((jnp.arange(R, dtype=jnp.int32) % NC) * NPAD)[:, None, None]
  src_p = src3.reshape(R * NS, ROWS, IDXW)
  srcg = (src3 + roff).reshape(R * NS, ROWS, IDXW)
  dst_p = dst3.reshape(R * NS, ROWS, IDXW)
  dstdeg = (dst3 + doff).reshape(R * NS, ROWS, IDXW)

  b_proj2 = b_proj.reshape(1, D)
  bias1_2 = bias1.reshape(1, D)
  bias2_2 = bias2.reshape(1, D)
  pw2 = prompt_w.reshape(1, D)

  deg = _sc_deg(dstdeg).reshape(R, NPAD)
  xw1 = _tc_proj(x_pad, W_proj, b_proj2, basis1, coeff1)
  S1 = _sc_scatter(xw1.reshape(R * NPAD, D), srcg, dst_p).reshape(R, NPAD, D)
  xw2 = _tc_conv(S1, deg, bias1_2, basis2, coeff2)
  S2 = _sc_scatter(xw2.reshape(R * NPAD, D), srcg, dst_p).reshape(R, NPAD, D)
  hp = _tc_prompt(S2, deg, bias2_2, pw2)
  Pf = _sc_final(hp, src_p, dst_p)
  return _tc_add(Pf.reshape(NC, NPAD, D))


# speedup vs baseline: 8.5144x; 1.1228x over previous
import functools

import jax
import jax.numpy as jnp
from jax import lax
from jax.experimental import pallas as pl
from jax.experimental.pallas import tpu as pltpu
from jax.experimental.pallas import tpu_sc as plsc

N = 10000
D = 128
E = 320000
R = 4
NB = 2
NC = 2
NS = 16
NPAD = 10240
TSPAN = NPAD // NS
EPT = E // R // NS
IDXW = 64
ROWS = 80
HROWS = 40
EPTP = ROWS * IDXW
PADE = EPTP - EPT
ZROWS = 64
BLK = 1024
GRID = NPAD // BLK

_sc_mesh = plsc.VectorSubcoreMesh(
    core_axis_name="c", subcore_axis_name="s", num_cores=NC, num_subcores=NS)


NBUF = 4


def _fill_rows(ref, rows, value):
  vec = jnp.full((16,), value, jnp.float32)

  def body(i, carry):
    for j in range(D // 16):
      ref[i, pl.ds(j * 16, 16)] = vec
    return carry

  lax.fori_loop(0, rows, body, 0)


def _edge_pass(xw, acc, srcg, dstg, row, src_v, dst_v, bufs, gsems, ssems):
  for h in range(ROWS // HROWS):
    pltpu.sync_copy(srcg.at[row, pl.ds(h * HROWS, HROWS)], src_v)
    pltpu.sync_copy(dstg.at[row, pl.ds(h * HROWS, HROWS)], dst_v)
    for b in range(NBUF):
      pltpu.async_copy(xw.at[src_v.at[b]], bufs[b], gsems[b])

    def body(i, carry):
      for b in range(NBUF):
        pltpu.make_async_copy(xw.at[pl.ds(0, IDXW)], bufs[b], gsems[b]).wait()
        pltpu.async_copy(bufs[b], acc.at[dst_v.at[NBUF * i + b]], ssems[b],
                         add=True)
      for b in range(NBUF):
        pltpu.make_async_copy(bufs[b], acc.at[pl.ds(0, IDXW)], ssems[b]).wait()

        @pl.when(i < HROWS // NBUF - 1)
        def _():
          pltpu.async_copy(xw.at[src_v.at[NBUF * (i + 1) + b]], bufs[b],
                           gsems[b])

      return carry

    lax.fori_loop(0, HROWS // NBUF, body, 0)


@functools.partial(
    pl.kernel,
    out_type=jax.ShapeDtypeStruct((R * NPAD, D), jnp.float32),
    mesh=_sc_mesh,
    scratch_types=[
        pltpu.VMEM_SHARED((NPAD, D), jnp.float32),
        pltpu.VMEM((HROWS, IDXW), jnp.int32),
        pltpu.VMEM((HROWS, IDXW), jnp.int32),
        [pltpu.VMEM((IDXW, D), jnp.float32)] * NBUF,
        [pltpu.SemaphoreType.DMA] * NBUF,
        [pltpu.SemaphoreType.DMA] * NBUF,
    ],
)
def _sc_scatter(xw, srcg, dstg, out, acc, src_v, dst_v, bufs, gsems,
                ssems):
  c = lax.axis_index("c")
  s = lax.axis_index("s")
  base = s * TSPAN
  for ri in range(R // NC):
    r = c * (R // NC) + ri
    _fill_rows(bufs[0], IDXW, 0.0)
    for k in range(TSPAN // IDXW):
      pltpu.sync_copy(bufs[0], acc.at[pl.ds(base + k * IDXW, IDXW)])
    plsc.subcore_barrier()
    _edge_pass(xw, acc, srcg, dstg, r * NS + s, src_v, dst_v, bufs, gsems,
               ssems)
    plsc.subcore_barrier()
    pltpu.sync_copy(acc.at[pl.ds(base, TSPAN)],
                    out.at[pl.ds(r * NPAD + base, TSPAN)])
    plsc.subcore_barrier()


@functools.partial(
    pl.kernel,
    out_type=jax.ShapeDtypeStruct((NC * NPAD, D), jnp.float32),
    mesh=_sc_mesh,
    scratch_types=[
        pltpu.VMEM_SHARED((NPAD, D), jnp.float32),
        pltpu.VMEM((HROWS, IDXW), jnp.int32),
        pltpu.VMEM((HROWS, IDXW), jnp.int32),
        [pltpu.VMEM((IDXW, D), jnp.float32)] * NBUF,
        [pltpu.SemaphoreType.DMA] * NBUF,
        [pltpu.SemaphoreType.DMA] * NBUF,
    ],
)
def _sc_final(hp, srcp, dstp, out, acc, src_v, dst_v, bufs, gsems,
              ssems):
  c = lax.axis_index("c")
  s = lax.axis_index("s")
  _fill_rows(bufs[0], IDXW, 0.0)
  base = s * TSPAN
  for k in range(TSPAN // IDXW):
    pltpu.sync_copy(bufs[0], acc.at[pl.ds(base + k * IDXW, IDXW)])
  plsc.subcore_barrier()
  for ri in range(R // NC):
    r = c * (R // NC) + ri
    _edge_pass(hp, acc, srcp, dstp, r * NS + s, src_v, dst_v, bufs, gsems,
               ssems)
  plsc.subcore_barrier()
  pltpu.sync_copy(acc.at[pl.ds(base, TSPAN)],
                  out.at[pl.ds(c * NPAD + base, TSPAN)])


@functools.partial(
    pl.kernel,
    out_type=jax.ShapeDtypeStruct((R * NPAD,), jnp.float32),
    mesh=_sc_mesh,
    scratch_types=[
        pltpu.VMEM_SHARED(((R // NC) * NPAD,), jnp.float32),
        pltpu.VMEM((ROWS, IDXW), jnp.int32),
        pltpu.VMEM((ROWS, IDXW), jnp.float32),
        pltpu.VMEM(((R // NC) * NPAD // NS,), jnp.float32),
    ],
)
def _sc_deg(dstdeg, out, accd, idx_v, ones_v, zb):
  c = lax.axis_index("c")
  s = lax.axis_index("s")
  span = (R // NC) * NPAD // NS
  vec1 = jnp.full((16,), 1.0, jnp.float32)
  vec0 = jnp.zeros((16,), jnp.float32)

  def fill(i, carry):
    for j in range(IDXW // 16):
      ones_v[i, pl.ds(j * 16, 16)] = vec1
    return carry

  lax.fori_loop(0, ROWS, fill, 0)

  def fillz(i, carry):
    zb[pl.ds(i * 16, 16)] = vec0
    return carry

  lax.fori_loop(0, span // 16, fillz, 0)
  pltpu.sync_copy(zb, accd.at[pl.ds(s * span, span)])
  plsc.subcore_barrier()
  for ri in range(R // NC):
    r = c * (R // NC) + ri
    pltpu.sync_copy(dstdeg.at[r * NS + s], idx_v)

    def chunk(j, carry):
      pltpu.sync_copy(ones_v.at[j], accd.at[idx_v.at[j]], add=True)
      return carry

    lax.fori_loop(0, ROWS, chunk, 0)
  plsc.subcore_barrier()
  pltpu.sync_copy(accd.at[pl.ds(s * span, span)],
                  out.at[pl.ds(c * (R // NC) * NPAD + s * span, span)])


def _tc_proj_body(x_ref, wp_ref, bp_ref, basis_ref, coeff_ref, out_ref):
  h0 = jnp.dot(x_ref[...], wp_ref[...],
               preferred_element_type=jnp.float32) + bp_ref[...]
  for r in range(R):
    w = basis_ref[0] * coeff_ref[r, 0] + basis_ref[1] * coeff_ref[r, 1]
    out_ref[r] = jnp.dot(h0, w, preferred_element_type=jnp.float32)


_tc_proj = pl.pallas_call(
    _tc_proj_body,
    grid=(GRID,),
    in_specs=[
        pl.BlockSpec((BLK, D), lambda i: (i, 0)),
        pl.BlockSpec((D, D), lambda i: (0, 0)),
        pl.BlockSpec((1, D), lambda i: (0, 0)),
        pl.BlockSpec((NB, D, D), lambda i: (0, 0, 0)),
        pl.BlockSpec((R, NB), lambda i: (0, 0)),
    ],
    out_specs=pl.BlockSpec((R, BLK, D), lambda i: (0, i, 0)),
    out_shape=jax.ShapeDtypeStruct((R, NPAD, D), jnp.float32),
)


def _norm_relu(s_ref, deg_ref, bias_ref):
  invd = 1.0 / jnp.maximum(deg_ref[...], 1.0)
  h = s_ref[0] * invd[0][:, None]
  for r in range(1, R):
    h = h + s_ref[r] * invd[r][:, None]
  return jnp.maximum(h + bias_ref[...], 0.0)


def _tc_conv_body(s_ref, deg_ref, bias_ref, basis_ref, coeff_ref, out_ref):
  h = _norm_relu(s_ref, deg_ref, bias_ref)
  for r in range(R):
    w = basis_ref[0] * coeff_ref[r, 0] + basis_ref[1] * coeff_ref[r, 1]
    out_ref[r] = jnp.dot(h, w, preferred_element_type=jnp.float32)


_tc_conv = pl.pallas_call(
    _tc_conv_body,
    grid=(GRID,),
    in_specs=[
        pl.BlockSpec((R, BLK, D), lambda i: (0, i, 0)),
        pl.BlockSpec((R, BLK), lambda i: (0, i)),
        pl.BlockSpec((1, D), lambda i: (0, 0)),
        pl.BlockSpec((NB, D, D), lambda i: (0, 0, 0)),
        pl.BlockSpec((R, NB), lambda i: (0, 0)),
    ],
    out_specs=pl.BlockSpec((R, BLK, D), lambda i: (0, i, 0)),
    out_shape=jax.ShapeDtypeStruct((R, NPAD, D), jnp.float32),
)


def _tc_prompt_body(s_ref, deg_ref, bias_ref, pw_ref, out_ref):
  h = _norm_relu(s_ref, deg_ref, bias_ref)
  z = h * pw_ref[...]
  out_ref[...] = jnp.where(z > 0, z, jnp.exp(jnp.minimum(z, 0.0)) - 1.0)


_tc_prompt = pl.pallas_call(
    _tc_prompt_body,
    grid=(GRID,),
    in_specs=[
        pl.BlockSpec((R, BLK, D), lambda i: (0, i, 0)),
        pl.BlockSpec((R, BLK), lambda i: (0, i)),
        pl.BlockSpec((1, D), lambda i: (0, 0)),
        pl.BlockSpec((1, D), lambda i: (0, 0)),
    ],
    out_specs=pl.BlockSpec((BLK, D), lambda i: (i, 0)),
    out_shape=jax.ShapeDtypeStruct((NPAD, D), jnp.float32),
)


def _tc_add_body(p_ref, out_ref):
  out_ref[...] = p_ref[0] + p_ref[1]


_tc_add = pl.pallas_call(
    _tc_add_body,
    grid=(GRID,),
    in_specs=[pl.BlockSpec((NC, BLK, D), lambda i: (0, i, 0))],
    out_specs=pl.BlockSpec((BLK, D), lambda i: (i, 0)),
    out_shape=jax.ShapeDtypeStruct((N, D), jnp.float32),
)


def kernel(x, edge_index, W_proj, b_proj, basis1, coeff1, bias1,
           basis2, coeff2, bias2, prompt_w):
  f32 = jnp.float32
  x_pad = jnp.concatenate([x, jnp.zeros((NPAD - N, D), f32)], axis=0)

  src = edge_index[0].astype(jnp.int32).reshape(R, NS, EPT)
  dst = edge_index[1].astype(jnp.int32).reshape(R, NS, EPT)
  padv = (jnp.arange(PADE, dtype=jnp.int32) % 8)
  pad_src = jnp.broadcast_to(padv, (R, NS, PADE))
  pad_dst = pad_src + N
  src3 = jnp.concatenate([src, pad_src], axis=2)
  dst3 = jnp.concatenate([dst, pad_dst], axis=2)
  roff = (jnp.arange(R, dtype=jnp.int32) * NPAD)[:, None, None]
  doff = ((jnp.arange(R, dtype=jnp.int32) % NC) * NPAD)[:, None, None]
  src_p = src3.reshape(R * NS, ROWS, IDXW)
  srcg = (src3 + roff).reshape(R * NS, ROWS, IDXW)
  dst_p = dst3.reshape(R * NS, ROWS, IDXW)
  dstdeg = (dst3 + doff).reshape(R * NS, ROWS, IDXW)

  b_proj2 = b_proj.reshape(1, D)
  bias1_2 = bias1.reshape(1, D)
  bias2_2 = bias2.reshape(1, D)
  pw2 = prompt_w.reshape(1, D)

  deg = _sc_deg(dstdeg).reshape(R, NPAD)
  xw1 = _tc_proj(x_pad, W_proj, b_proj2, basis1, coeff1)
  S1 = _sc_scatter(xw1.reshape(R * NPAD, D), srcg, dst_p).reshape(R, NPAD, D)
  xw2 = _tc_conv(S1, deg, bias1_2, basis2, coeff2)
  S2 = _sc_scatter(xw2.reshape(R * NPAD, D), srcg, dst_p).reshape(R, NPAD, D)
  hp = _tc_prompt(S2, deg, bias2_2, pw2)
  Pf = _sc_final(hp, src_p, dst_p)
  return _tc_add(Pf.reshape(NC, NPAD, D))


# speedup vs baseline: 8.5229x; 1.0010x over previous
import functools

import jax
import jax.numpy as jnp
from jax import lax
from jax.experimental import pallas as pl
from jax.experimental.pallas import tpu as pltpu
from jax.experimental.pallas import tpu_sc as plsc

N = 10000
D = 128
E = 320000
R = 4
NB = 2
NC = 2
NS = 16
NPAD = 10240
TSPAN = NPAD // NS
EPT = E // R // NS
IDXW = 64
ROWS = 80
HROWS = 40
EPTP = ROWS * IDXW
PADE = EPTP - EPT
ZROWS = 64
BLK = 1024
GRID = NPAD // BLK

_sc_mesh = plsc.VectorSubcoreMesh(
    core_axis_name="c", subcore_axis_name="s", num_cores=NC, num_subcores=NS)


NBUF = 4


def _fill_rows(ref, rows, value):
  vec = jnp.full((16,), value, jnp.float32)

  def body(i, carry):
    for j in range(D // 16):
      ref[i, pl.ds(j * 16, 16)] = vec
    return carry

  lax.fori_loop(0, rows, body, 0)


def _edge_pass(xw, acc, srcg, dstg, row, src_v, dst_v, bufs, gsems, ssems):
  for h in range(ROWS // HROWS):
    pltpu.sync_copy(srcg.at[row, pl.ds(h * HROWS, HROWS)], src_v)
    pltpu.sync_copy(dstg.at[row, pl.ds(h * HROWS, HROWS)], dst_v)
    for b in range(NBUF):
      pltpu.async_copy(xw.at[src_v.at[b]], bufs[b], gsems[b])

    def body(i, carry):
      for b in range(NBUF):
        pltpu.make_async_copy(xw.at[pl.ds(0, IDXW)], bufs[b], gsems[b]).wait()
        pltpu.async_copy(bufs[b], acc.at[dst_v.at[NBUF * i + b]], ssems[b],
                         add=True)
      for b in range(NBUF):
        pltpu.make_async_copy(bufs[b], acc.at[pl.ds(0, IDXW)], ssems[b]).wait()

        @pl.when(i < HROWS // NBUF - 1)
        def _():
          pltpu.async_copy(xw.at[src_v.at[NBUF * (i + 1) + b]], bufs[b],
                           gsems[b])

      return carry

    lax.fori_loop(0, HROWS // NBUF, body, 0)


@functools.partial(
    pl.kernel,
    out_type=jax.ShapeDtypeStruct((R * NPAD, D), jnp.float32),
    mesh=_sc_mesh,
    scratch_types=[
        pltpu.VMEM_SHARED((NPAD, D), jnp.float32),
        pltpu.VMEM((HROWS, IDXW), jnp.int32),
        pltpu.VMEM((HROWS, IDXW), jnp.int32),
        [pltpu.VMEM((IDXW, D), jnp.float32)] * NBUF,
        [pltpu.SemaphoreType.DMA] * NBUF,
        [pltpu.SemaphoreType.DMA] * NBUF,
    ],
)
def _sc_scatter(xw, srcg, dstg, out, acc, src_v, dst_v, bufs, gsems,
                ssems):
  c = lax.axis_index("c")
  s = lax.axis_index("s")
  base = s * TSPAN
  _fill_rows(bufs[0], IDXW, 0.0)
  for k in range(TSPAN // IDXW):
    pltpu.sync_copy(bufs[0], acc.at[pl.ds(base + k * IDXW, IDXW)])
  for ri in range(R // NC):
    r = c * (R // NC) + ri
    plsc.subcore_barrier()
    _edge_pass(xw, acc, srcg, dstg, r * NS + s, src_v, dst_v, bufs, gsems,
               ssems)
    plsc.subcore_barrier()
    pltpu.sync_copy(acc.at[pl.ds(base, TSPAN)],
                    out.at[pl.ds(r * NPAD + base, TSPAN)])
    if ri < R // NC - 1:
      _fill_rows(bufs[0], IDXW, 0.0)
      for k in range(TSPAN // IDXW):
        pltpu.sync_copy(bufs[0], acc.at[pl.ds(base + k * IDXW, IDXW)])


@functools.partial(
    pl.kernel,
    out_type=jax.ShapeDtypeStruct((NC * NPAD, D), jnp.float32),
    mesh=_sc_mesh,
    scratch_types=[
        pltpu.VMEM_SHARED((NPAD, D), jnp.float32),
        pltpu.VMEM((HROWS, IDXW), jnp.int32),
        pltpu.VMEM((HROWS, IDXW), jnp.int32),
        [pltpu.VMEM((IDXW, D), jnp.float32)] * NBUF,
        [pltpu.SemaphoreType.DMA] * NBUF,
        [pltpu.SemaphoreType.DMA] * NBUF,
    ],
)
def _sc_final(hp, srcp, dstp, out, acc, src_v, dst_v, bufs, gsems,
              ssems):
  c = lax.axis_index("c")
  s = lax.axis_index("s")
  _fill_rows(bufs[0], IDXW, 0.0)
  base = s * TSPAN
  for k in range(TSPAN // IDXW):
    pltpu.sync_copy(bufs[0], acc.at[pl.ds(base + k * IDXW, IDXW)])
  plsc.subcore_barrier()
  for ri in range(R // NC):
    r = c * (R // NC) + ri
    _edge_pass(hp, acc, srcp, dstp, r * NS + s, src_v, dst_v, bufs, gsems,
               ssems)
  plsc.subcore_barrier()
  pltpu.sync_copy(acc.at[pl.ds(base, TSPAN)],
                  out.at[pl.ds(c * NPAD + base, TSPAN)])


@functools.partial(
    pl.kernel,
    out_type=jax.ShapeDtypeStruct((R * NPAD,), jnp.float32),
    mesh=_sc_mesh,
    scratch_types=[
        pltpu.VMEM_SHARED(((R // NC) * NPAD,), jnp.float32),
        pltpu.VMEM((ROWS, IDXW), jnp.int32),
        pltpu.VMEM((ROWS, IDXW), jnp.float32),
        pltpu.VMEM(((R // NC) * NPAD // NS,), jnp.float32),
    ],
)
def _sc_deg(dstdeg, out, accd, idx_v, ones_v, zb):
  c = lax.axis_index("c")
  s = lax.axis_index("s")
  span = (R // NC) * NPAD // NS
  vec1 = jnp.full((16,), 1.0, jnp.float32)
  vec0 = jnp.zeros((16,), jnp.float32)

  def fill(i, carry):
    for j in range(IDXW // 16):
      ones_v[i, pl.ds(j * 16, 16)] = vec1
    return carry

  lax.fori_loop(0, ROWS, fill, 0)

  def fillz(i, carry):
    zb[pl.ds(i * 16, 16)] = vec0
    return carry

  lax.fori_loop(0, span // 16, fillz, 0)
  pltpu.sync_copy(zb, accd.at[pl.ds(s * span, span)])
  plsc.subcore_barrier()
  for ri in range(R // NC):
    r = c * (R // NC) + ri
    pltpu.sync_copy(dstdeg.at[r * NS + s], idx_v)

    def chunk(j, carry):
      pltpu.sync_copy(ones_v.at[j], accd.at[idx_v.at[j]], add=True)
      return carry

    lax.fori_loop(0, ROWS, chunk, 0)
  plsc.subcore_barrier()
  pltpu.sync_copy(accd.at[pl.ds(s * span, span)],
                  out.at[pl.ds(c * (R // NC) * NPAD + s * span, span)])


def _tc_proj_body(x_ref, wp_ref, bp_ref, basis_ref, coeff_ref, out_ref):
  h0 = jnp.dot(x_ref[...], wp_ref[...],
               preferred_element_type=jnp.float32) + bp_ref[...]
  for r in range(R):
    w = basis_ref[0] * coeff_ref[r, 0] + basis_ref[1] * coeff_ref[r, 1]
    out_ref[r] = jnp.dot(h0, w, preferred_element_type=jnp.float32)


_tc_proj = pl.pallas_call(
    _tc_proj_body,
    grid=(GRID,),
    in_specs=[
        pl.BlockSpec((BLK, D), lambda i: (i, 0)),
        pl.BlockSpec((D, D), lambda i: (0, 0)),
        pl.BlockSpec((1, D), lambda i: (0, 0)),
        pl.BlockSpec((NB, D, D), lambda i: (0, 0, 0)),
        pl.BlockSpec((R, NB), lambda i: (0, 0)),
    ],
    out_specs=pl.BlockSpec((R, BLK, D), lambda i: (0, i, 0)),
    out_shape=jax.ShapeDtypeStruct((R, NPAD, D), jnp.float32),
)


def _norm_relu(s_ref, deg_ref, bias_ref):
  invd = 1.0 / jnp.maximum(deg_ref[...], 1.0)
  h = s_ref[0] * invd[0][:, None]
  for r in range(1, R):
    h = h + s_ref[r] * invd[r][:, None]
  return jnp.maximum(h + bias_ref[...], 0.0)


def _tc_conv_body(s_ref, deg_ref, bias_ref, basis_ref, coeff_ref, out_ref):
  h = _norm_relu(s_ref, deg_ref, bias_ref)
  for r in range(R):
    w = basis_ref[0] * coeff_ref[r, 0] + basis_ref[1] * coeff_ref[r, 1]
    out_ref[r] = jnp.dot(h, w, preferred_element_type=jnp.float32)


_tc_conv = pl.pallas_call(
    _tc_conv_body,
    grid=(GRID,),
    in_specs=[
        pl.BlockSpec((R, BLK, D), lambda i: (0, i, 0)),
        pl.BlockSpec((R, BLK), lambda i: (0, i)),
        pl.BlockSpec((1, D), lambda i: (0, 0)),
        pl.BlockSpec((NB, D, D), lambda i: (0, 0, 0)),
        pl.BlockSpec((R, NB), lambda i: (0, 0)),
    ],
    out_specs=pl.BlockSpec((R, BLK, D), lambda i: (0, i, 0)),
    out_shape=jax.ShapeDtypeStruct((R, NPAD, D), jnp.float32),
)


def _tc_prompt_body(s_ref, deg_ref, bias_ref, pw_ref, out_ref):
  h = _norm_relu(s_ref, deg_ref, bias_ref)
  z = h * pw_ref[...]
  out_ref[...] = jnp.where(z > 0, z, jnp.exp(jnp.minimum(z, 0.0)) - 1.0)


_tc_prompt = pl.pallas_call(
    _tc_prompt_body,
    grid=(GRID,),
    in_specs=[
        pl.BlockSpec((R, BLK, D), lambda i: (0, i, 0)),
        pl.BlockSpec((R, BLK), lambda i: (0, i)),
        pl.BlockSpec((1, D), lambda i: (0, 0)),
        pl.BlockSpec((1, D), lambda i: (0, 0)),
    ],
    out_specs=pl.BlockSpec((BLK, D), lambda i: (i, 0)),
    out_shape=jax.ShapeDtypeStruct((NPAD, D), jnp.float32),
)


def _tc_add_body(p_ref, out_ref):
  out_ref[...] = p_ref[0] + p_ref[1]


_tc_add = pl.pallas_call(
    _tc_add_body,
    grid=(GRID,),
    in_specs=[pl.BlockSpec((NC, BLK, D), lambda i: (0, i, 0))],
    out_specs=pl.BlockSpec((BLK, D), lambda i: (i, 0)),
    out_shape=jax.ShapeDtypeStruct((N, D), jnp.float32),
)


def kernel(x, edge_index, W_proj, b_proj, basis1, coeff1, bias1,
           basis2, coeff2, bias2, prompt_w):
  f32 = jnp.float32
  x_pad = jnp.concatenate([x, jnp.zeros((NPAD - N, D), f32)], axis=0)

  src = edge_index[0].astype(jnp.int32).reshape(R, NS, EPT)
  dst = edge_index[1].astype(jnp.int32).reshape(R, NS, EPT)
  padv = (jnp.arange(PADE, dtype=jnp.int32) % 8)
  pad_src = jnp.broadcast_to(padv, (R, NS, PADE))
  pad_dst = pad_src + N
  src3 = jnp.concatenate([src, pad_src], axis=2)
  dst3 = jnp.concatenate([dst, pad_dst], axis=2)
  roff = (jnp.arange(R, dtype=jnp.int32) * NPAD)[:, None, None]
  doff = ((jnp.arange(R, dtype=jnp.int32) % NC) * NPAD)[:, None, None]
  src_p = src3.reshape(R * NS, ROWS, IDXW)
  srcg = (src3 + roff).reshape(R * NS, ROWS, IDXW)
  dst_p = dst3.reshape(R * NS, ROWS, IDXW)
  dstdeg = (dst3 + doff).reshape(R * NS, ROWS, IDXW)

  b_proj2 = b_proj.reshape(1, D)
  bias1_2 = bias1.reshape(1, D)
  bias2_2 = bias2.reshape(1, D)
  pw2 = prompt_w.reshape(1, D)

  deg = _sc_deg(dstdeg).reshape(R, NPAD)
  xw1 = _tc_proj(x_pad, W_proj, b_proj2, basis1, coeff1)
  S1 = _sc_scatter(xw1.reshape(R * NPAD, D), srcg, dst_p).reshape(R, NPAD, D)
  xw2 = _tc_conv(S1, deg, bias1_2, basis2, coeff2)
  S2 = _sc_scatter(xw2.reshape(R * NPAD, D), srcg, dst_p).reshape(R, NPAD, D)
  hp = _tc_prompt(S2, deg, bias2_2, pw2)
  Pf = _sc_final(hp, src_p, dst_p)
  return _tc_add(Pf.reshape(NC, NPAD, D))
